# Initial kernel scaffold; baseline (speedup 1.0000x reference)
import jax
import jax.numpy as jnp
from jax.experimental import pallas as pl


def kernel(input, params):
    raise NotImplementedError("write your pallas kernel here")


# speedup vs baseline: 8.7660x; 8.7660x over previous
import functools

import jax
import jax.numpy as jnp
from jax import lax
from jax.experimental import pallas as pl
from jax.experimental.pallas import tpu as pltpu
from jax.experimental.pallas import tpu_sc as plsc

_NPOINT = (2048, 1024, 512)
_RADIUS = (0.2, 0.4, 0.8)
_NSAMPLE = (64, 32, 16)


def _fps_body(x_ref, y_ref, z_ref, ind_ref, nx_ref, ny_ref, nz_ref, *, S, N):
    nd8 = N // 8
    X = x_ref[0]
    Y = y_ref[0]
    Z = z_ref[0]
    row = lax.broadcasted_iota(jnp.int32, (8, nd8), 0)
    col = lax.broadcasted_iota(jnp.int32, (8, nd8), 1)
    iota = row * nd8 + col

    def step(i, carry):
        dists, far = carry
        cm = iota == far
        cx = jnp.sum(jnp.where(cm, X, 0.0))
        cy = jnp.sum(jnp.where(cm, Y, 0.0))
        cz = jnp.sum(jnp.where(cm, Z, 0.0))
        dx = X - cx
        dy = Y - cy
        dz = Z - cz
        d = (dx * dx + dy * dy) + dz * dz
        dists = jnp.minimum(dists, d)
        ind_ref[0, pl.ds(i, 1), :] = far.reshape(1, 1)
        nx_ref[0, pl.ds(i, 1), :] = cx.reshape(1, 1)
        ny_ref[0, pl.ds(i, 1), :] = cy.reshape(1, 1)
        nz_ref[0, pl.ds(i, 1), :] = cz.reshape(1, 1)
        m = jnp.max(dists)
        far2 = jnp.min(jnp.where(dists == m, iota, N))
        return dists, far2

    d0 = jnp.full((8, nd8), 1e10, jnp.float32)
    lax.fori_loop(0, S, step, (d0, jnp.zeros((), jnp.int32)))


def _fps(xp, yp, zp, S):
    B, _, nd8 = xp.shape
    N = 8 * nd8
    body = functools.partial(_fps_body, S=S, N=N)
    in_spec = pl.BlockSpec((1, 8, nd8), lambda b: (b, 0, 0))
    out_spec = pl.BlockSpec((1, S, 1), lambda b: (b, 0, 0))
    out_shape = [jax.ShapeDtypeStruct((B, S, 1), jnp.int32)] + [
        jax.ShapeDtypeStruct((B, S, 1), jnp.float32)
    ] * 3
    return pl.pallas_call(
        body,
        grid=(B,),
        in_specs=[in_spec] * 3,
        out_specs=[out_spec] * 4,
        out_shape=out_shape,
        compiler_params=pltpu.CompilerParams(
            dimension_semantics=("parallel",)
        ),
    )(xp, yp, zp)


def _ball_body(qx_ref, qy_ref, qz_ref, xl_ref, yl_ref, zl_ref, idx_ref, *,
               K, N, r2, sblk):
    b = pl.program_id(0)
    qx = qx_ref[0]
    qy = qy_ref[0]
    qz = qz_ref[0]
    X = xl_ref[0]
    Y = yl_ref[0]
    Z = zl_ref[0]
    dx = qx - X
    dy = qy - Y
    dz = qz - Z
    d2 = (dx * dx + dy * dy) + dz * dz
    iota = lax.broadcasted_iota(jnp.int32, (sblk, N), 1)
    order = jnp.where(d2 < r2, iota, N)
    prev = jnp.full((sblk, 1), -1, jnp.int32)
    first = None
    cols = []
    for k in range(K):
        cur = jnp.min(jnp.where(order > prev, order, N), axis=1, keepdims=True)
        if k == 0:
            first = jnp.where(cur < N, cur, 0)
        cols.append(jnp.where(cur < N, cur, first) + b * N)
        prev = cur
    idx_ref[0] = jnp.concatenate(cols, axis=1)


def _ball(nx, ny, nz, xl, yl, zl, K, radius, sblk):
    B, S, _ = nx.shape
    N = xl.shape[2]
    body = functools.partial(_ball_body, K=K, N=N, r2=radius * radius,
                             sblk=sblk)
    q_spec = pl.BlockSpec((1, sblk, 1), lambda b, s: (b, s, 0))
    l_spec = pl.BlockSpec((1, 1, N), lambda b, s: (b, 0, 0))
    out_spec = pl.BlockSpec((1, sblk, K), lambda b, s: (b, s, 0))
    return pl.pallas_call(
        body,
        grid=(B, S // sblk),
        in_specs=[q_spec] * 3 + [l_spec] * 3,
        out_specs=out_spec,
        out_shape=jax.ShapeDtypeStruct((B, S, K), jnp.int32),
        compiler_params=pltpu.CompilerParams(
            dimension_semantics=("parallel", "parallel")
        ),
    )(nx, ny, nz, xl, yl, zl)


def _sc_gather(table, idx):
    V, D = table.shape
    Bt = idx.shape[0]
    info = plsc.get_sparse_core_info()
    nw = info.num_cores * info.num_subcores
    b_per_w = Bt // nw
    cap = max(8, (65536 // D) // 8 * 8)
    chunk = b_per_w
    while chunk > cap:
        chunk //= 2
    nchunk = b_per_w // chunk
    mesh = plsc.VectorSubcoreMesh(core_axis_name="c", subcore_axis_name="s")

    @functools.partial(
        pl.kernel,
        mesh=mesh,
        out_type=jax.ShapeDtypeStruct((Bt, D), jnp.float32),
        scratch_types=[
            pltpu.VMEM((chunk,), jnp.int32),
            pltpu.VMEM((chunk, D), jnp.float32),
            pltpu.SemaphoreType.DMA,
        ],
    )
    def k(table_hbm, idx_hbm, out_hbm, idx_v, rows_v, sem):
        wid = lax.axis_index("s") * info.num_cores + lax.axis_index("c")
        base = wid * b_per_w
        for i in range(nchunk):
            off = base + i * chunk
            pltpu.sync_copy(idx_hbm.at[pl.ds(off, chunk)], idx_v)
            pltpu.async_copy(table_hbm.at[idx_v], rows_v, sem).wait()
            pltpu.sync_copy(rows_v, out_hbm.at[pl.ds(off, chunk)])

    return k(table, idx)


def _mlp_body(g_ref, q_ref, wg_ref, b1_ref, wsh_ref, w2_ref, b2_ref, w3_ref,
              b3_ref, out_ref, *, K, sblk):
    hp = lax.Precision.HIGHEST
    g = g_ref[...]
    A = jnp.dot(g, wg_ref[...], preferred_element_type=jnp.float32,
                precision=hp)
    shift = jnp.dot(q_ref[...], wsh_ref[...],
                    preferred_element_type=jnp.float32, precision=hp)
    c1 = A.shape[1]
    A = (A.reshape(sblk, K, c1) - shift.reshape(sblk, 1, c1)).reshape(
        sblk * K, c1)
    h = jnp.maximum(A + b1_ref[...], 0.0)
    h = jnp.maximum(
        jnp.dot(h, w2_ref[...], preferred_element_type=jnp.float32,
                precision=hp) + b2_ref[...], 0.0)
    h = jnp.maximum(
        jnp.dot(h, w3_ref[...], preferred_element_type=jnp.float32,
                precision=hp) + b3_ref[...], 0.0)
    cout = h.shape[1]
    out_ref[...] = jnp.max(h.reshape(sblk, K, cout), axis=1)


def _mlp(g, q8, wg, b1, wsh, w2, b2, w3, b3, K, sblk):
    BT, D = g.shape
    BS = BT // K
    c3 = w3.shape[1]
    body = functools.partial(_mlp_body, K=K, sblk=sblk)
    full = lambda a: pl.BlockSpec(a.shape, lambda i: (0,) * a.ndim)
    return pl.pallas_call(
        body,
        grid=(BS // sblk,),
        in_specs=[
            pl.BlockSpec((sblk * K, D), lambda i: (i, 0)),
            pl.BlockSpec((sblk, 8), lambda i: (i, 0)),
            full(wg), full(b1), full(wsh), full(w2), full(b2), full(w3),
            full(b3),
        ],
        out_specs=pl.BlockSpec((sblk, c3), lambda i: (i, 0)),
        out_shape=jax.ShapeDtypeStruct((BS, c3), jnp.float32),
        compiler_params=pltpu.CompilerParams(
            dimension_semantics=("parallel",)
        ),
    )(g, q8, wg, b1, wsh, w2, b2, w3, b3)


def _fold_bn(W, bb, g, be, mu, var):
    s = g * lax.rsqrt(var + 1e-5)
    return W.T * s[None, :], bb * s + be - mu * s


def _pad_cols(a, d):
    return jnp.pad(a, ((0, 0), (0, d - a.shape[1])))


def _round128(n):
    return (n + 127) // 128 * 128


def kernel(input, params):
    B, _, N0 = input.shape
    xyzf = jnp.transpose(input, (0, 2, 1))
    xyz = xyzf[..., :3]
    feats = xyzf[..., 3:]

    outs = []
    old_ind = None
    for i in range(3):
        S, K, radius = _NPOINT[i], _NSAMPLE[i], _RADIUS[i]
        N = xyz.shape[1]
        C = feats.shape[2]

        xp = xyz[..., 0].reshape(B, 8, N // 8)
        yp = xyz[..., 1].reshape(B, 8, N // 8)
        zp = xyz[..., 2].reshape(B, 8, N // 8)
        ind, nx, ny, nz = _fps(xp, yp, zp, S)

        xl = xyz[..., 0].reshape(B, 1, N)
        yl = xyz[..., 1].reshape(B, 1, N)
        zl = xyz[..., 2].reshape(B, 1, N)
        idx = _ball(nx, ny, nz, xl, yl, zl, K, radius, min(128, S))

        D = _round128(3 + C)
        table = _pad_cols(
            jnp.concatenate([xyz, feats], axis=2).reshape(B * N, 3 + C), D)
        grouped = _sc_gather(table, idx.reshape(B * S * K))

        (w1, b1) = _fold_bn(*params[i][0])
        (w2, b2) = _fold_bn(*params[i][1])
        (w3, b3) = _fold_bn(*params[i][2])
        wg = jnp.concatenate([w1[:3] / radius, w1[3:]], axis=0)
        wg = jnp.pad(wg, ((0, D - wg.shape[0]), (0, 0)))
        wsh = jnp.pad(w1[:3] / radius, ((0, 5), (0, 0)))

        new_xyz = jnp.concatenate([nx, ny, nz], axis=2)
        q8 = jnp.pad(new_xyz, ((0, 0), (0, 0), (0, 5))).reshape(B * S, 8)
        feat_flat = _mlp(grouped, q8, wg, b1[None], wsh, w2, b2[None], w3,
                         b3[None], K, min(128, S))

        ind2 = ind[..., 0]
        if i == 0:
            old_ind = ind2
        else:
            prev_s = old_ind.shape[1]
            tab = jnp.broadcast_to(
                old_ind.astype(jnp.float32).reshape(B * prev_s, 1),
                (B * prev_s, 128))
            off = (jnp.arange(B, dtype=jnp.int32) * prev_s)[:, None]
            comp = _sc_gather(jnp.asarray(tab), (ind2 + off).reshape(B * S))
            old_ind = comp[:, 0].reshape(B, S).astype(jnp.int32)

        outs.append(new_xyz)
        outs.append(old_ind.astype(jnp.int64))

        xyz = new_xyz
        feats = feat_flat.reshape(B, S, feat_flat.shape[1])

    final_feats = jnp.transpose(feats, (0, 2, 1))
    return (xyz, final_feats) + tuple(outs)


# speedup vs baseline: 9.9517x; 1.1353x over previous
import functools

import jax
import jax.numpy as jnp
from jax import lax
from jax.experimental import pallas as pl
from jax.experimental.pallas import tpu as pltpu
from jax.experimental.pallas import tpu_sc as plsc

_NPOINT = (2048, 1024, 512)
_RADIUS = (0.2, 0.4, 0.8)
_NSAMPLE = (64, 32, 16)


def _fps_body(x_ref, y_ref, z_ref, xs_ref, ys_ref, zs_ref, ind_ref, nx_ref,
              ny_ref, nz_ref, *, S, N):
    nd8 = N // 8
    X = x_ref[0]
    Y = y_ref[0]
    Z = z_ref[0]
    row = lax.broadcasted_iota(jnp.int32, (8, nd8), 0)
    col = lax.broadcasted_iota(jnp.int32, (8, nd8), 1)
    iota = row * nd8 + col

    def step(i, carry):
        dists, far = carry
        cx = xs_ref[0, 0, far]
        cy = ys_ref[0, 0, far]
        cz = zs_ref[0, 0, far]
        dx = X - cx
        dy = Y - cy
        dz = Z - cz
        d = (dx * dx + dy * dy) + dz * dz
        dists = jnp.minimum(dists, d)
        ind_ref[0, pl.ds(i, 1), :] = far.reshape(1, 1)
        nx_ref[0, pl.ds(i, 1), :] = cx.reshape(1, 1)
        ny_ref[0, pl.ds(i, 1), :] = cy.reshape(1, 1)
        nz_ref[0, pl.ds(i, 1), :] = cz.reshape(1, 1)
        m = jnp.max(dists)
        far2 = jnp.min(jnp.where(dists == m, iota, N))
        return dists, far2

    d0 = jnp.full((8, nd8), 1e10, jnp.float32)
    lax.fori_loop(0, S, step, (d0, jnp.zeros((), jnp.int32)))


def _fps(xp, yp, zp, S):
    B, _, nd8 = xp.shape
    N = 8 * nd8
    body = functools.partial(_fps_body, S=S, N=N)
    in_spec = pl.BlockSpec((1, 8, nd8), lambda b: (b, 0, 0))
    s_spec = pl.BlockSpec((1, 1, N), lambda b: (b, 0, 0),
                          memory_space=pltpu.SMEM)
    out_spec = pl.BlockSpec((1, S, 1), lambda b: (b, 0, 0))
    out_shape = [jax.ShapeDtypeStruct((B, S, 1), jnp.int32)] + [
        jax.ShapeDtypeStruct((B, S, 1), jnp.float32)
    ] * 3
    flat = [a.reshape(B, 1, N) for a in (xp, yp, zp)]
    return pl.pallas_call(
        body,
        grid=(B,),
        in_specs=[in_spec] * 3 + [s_spec] * 3,
        out_specs=[out_spec] * 4,
        out_shape=out_shape,
        compiler_params=pltpu.CompilerParams(
            dimension_semantics=("parallel",)
        ),
    )(xp, yp, zp, *flat)


def _ball_body(qx_ref, qy_ref, qz_ref, xl_ref, yl_ref, zl_ref, idx_ref, *,
               K, N, r2, sblk):
    b = pl.program_id(0)
    qx = qx_ref[0]
    qy = qy_ref[0]
    qz = qz_ref[0]
    X = xl_ref[0]
    Y = yl_ref[0]
    Z = zl_ref[0]
    dx = qx - X
    dy = qy - Y
    dz = qz - Z
    d2 = (dx * dx + dy * dy) + dz * dz
    iota = lax.broadcasted_iota(jnp.int32, (sblk, N), 1)
    order = jnp.where(d2 < r2, iota, N)
    prev = jnp.full((sblk, 1), -1, jnp.int32)
    first = None
    cols = []
    for k in range(K):
        cur = jnp.min(jnp.where(order > prev, order, N), axis=1, keepdims=True)
        if k == 0:
            first = jnp.where(cur < N, cur, 0)
        cols.append(jnp.where(cur < N, cur, first) + b * N)
        prev = cur
    idx_ref[0] = jnp.concatenate(cols, axis=1)


def _ball(nx, ny, nz, xl, yl, zl, K, radius, sblk):
    B, S, _ = nx.shape
    N = xl.shape[2]
    body = functools.partial(_ball_body, K=K, N=N, r2=radius * radius,
                             sblk=sblk)
    q_spec = pl.BlockSpec((1, sblk, 1), lambda b, s: (b, s, 0))
    l_spec = pl.BlockSpec((1, 1, N), lambda b, s: (b, 0, 0))
    out_spec = pl.BlockSpec((1, sblk, K), lambda b, s: (b, s, 0))
    return pl.pallas_call(
        body,
        grid=(B, S // sblk),
        in_specs=[q_spec] * 3 + [l_spec] * 3,
        out_specs=out_spec,
        out_shape=jax.ShapeDtypeStruct((B, S, K), jnp.int32),
        compiler_params=pltpu.CompilerParams(
            dimension_semantics=("parallel", "parallel")
        ),
    )(nx, ny, nz, xl, yl, zl)


def _sc_gather(table, idx):
    V, D = table.shape
    Bt = idx.shape[0]
    info = plsc.get_sparse_core_info()
    nw = info.num_cores * info.num_subcores
    b_per_w = Bt // nw
    cap = max(8, (65536 // D) // 8 * 8)
    chunk = b_per_w
    while chunk > cap:
        chunk //= 2
    nchunk = b_per_w // chunk
    mesh = plsc.VectorSubcoreMesh(core_axis_name="c", subcore_axis_name="s")

    @functools.partial(
        pl.kernel,
        mesh=mesh,
        out_type=jax.ShapeDtypeStruct((Bt, D), jnp.float32),
        scratch_types=[
            pltpu.VMEM((chunk,), jnp.int32),
            pltpu.VMEM((chunk, D), jnp.float32),
            pltpu.SemaphoreType.DMA,
        ],
    )
    def k(table_hbm, idx_hbm, out_hbm, idx_v, rows_v, sem):
        wid = lax.axis_index("s") * info.num_cores + lax.axis_index("c")
        base = wid * b_per_w
        for i in range(nchunk):
            off = base + i * chunk
            pltpu.sync_copy(idx_hbm.at[pl.ds(off, chunk)], idx_v)
            pltpu.async_copy(table_hbm.at[idx_v], rows_v, sem).wait()
            pltpu.sync_copy(rows_v, out_hbm.at[pl.ds(off, chunk)])

    return k(table, idx)


def _mlp_body(g_ref, q_ref, wg_ref, b1_ref, wsh_ref, w2_ref, b2_ref, w3_ref,
              b3_ref, out_ref, *, K, sblk):
    hp = lax.Precision.HIGHEST
    g = g_ref[...]
    A = jnp.dot(g, wg_ref[...], preferred_element_type=jnp.float32,
                precision=hp)
    shift = jnp.dot(q_ref[...], wsh_ref[...],
                    preferred_element_type=jnp.float32, precision=hp)
    c1 = A.shape[1]
    A = (A.reshape(sblk, K, c1) - shift.reshape(sblk, 1, c1)).reshape(
        sblk * K, c1)
    h = jnp.maximum(A + b1_ref[...], 0.0)
    h = jnp.maximum(
        jnp.dot(h, w2_ref[...], preferred_element_type=jnp.float32,
                precision=hp) + b2_ref[...], 0.0)
    h = jnp.maximum(
        jnp.dot(h, w3_ref[...], preferred_element_type=jnp.float32,
                precision=hp) + b3_ref[...], 0.0)
    cout = h.shape[1]
    out_ref[...] = jnp.max(h.reshape(sblk, K, cout), axis=1)


def _mlp(g, q8, wg, b1, wsh, w2, b2, w3, b3, K, sblk):
    BT, D = g.shape
    BS = BT // K
    c3 = w3.shape[1]
    body = functools.partial(_mlp_body, K=K, sblk=sblk)
    full = lambda a: pl.BlockSpec(a.shape, lambda i: (0,) * a.ndim)
    return pl.pallas_call(
        body,
        grid=(BS // sblk,),
        in_specs=[
            pl.BlockSpec((sblk * K, D), lambda i: (i, 0)),
            pl.BlockSpec((sblk, 8), lambda i: (i, 0)),
            full(wg), full(b1), full(wsh), full(w2), full(b2), full(w3),
            full(b3),
        ],
        out_specs=pl.BlockSpec((sblk, c3), lambda i: (i, 0)),
        out_shape=jax.ShapeDtypeStruct((BS, c3), jnp.float32),
        compiler_params=pltpu.CompilerParams(
            dimension_semantics=("parallel",)
        ),
    )(g, q8, wg, b1, wsh, w2, b2, w3, b3)


def _fold_bn(W, bb, g, be, mu, var):
    s = g * lax.rsqrt(var + 1e-5)
    return W.T * s[None, :], bb * s + be - mu * s


def _pad_cols(a, d):
    return jnp.pad(a, ((0, 0), (0, d - a.shape[1])))


def _round128(n):
    return (n + 127) // 128 * 128


def kernel(input, params):
    B, _, N0 = input.shape
    xyzf = jnp.transpose(input, (0, 2, 1))
    xyz = xyzf[..., :3]
    feats = xyzf[..., 3:]

    outs = []
    old_ind = None
    for i in range(3):
        S, K, radius = _NPOINT[i], _NSAMPLE[i], _RADIUS[i]
        N = xyz.shape[1]
        C = feats.shape[2]

        xp = xyz[..., 0].reshape(B, 8, N // 8)
        yp = xyz[..., 1].reshape(B, 8, N // 8)
        zp = xyz[..., 2].reshape(B, 8, N // 8)
        ind, nx, ny, nz = _fps(xp, yp, zp, S)

        xl = xyz[..., 0].reshape(B, 1, N)
        yl = xyz[..., 1].reshape(B, 1, N)
        zl = xyz[..., 2].reshape(B, 1, N)
        idx = _ball(nx, ny, nz, xl, yl, zl, K, radius, min(128, S))

        D = _round128(3 + C)
        table = _pad_cols(
            jnp.concatenate([xyz, feats], axis=2).reshape(B * N, 3 + C), D)
        grouped = _sc_gather(table, idx.reshape(B * S * K))

        (w1, b1) = _fold_bn(*params[i][0])
        (w2, b2) = _fold_bn(*params[i][1])
        (w3, b3) = _fold_bn(*params[i][2])
        wg = jnp.concatenate([w1[:3] / radius, w1[3:]], axis=0)
        wg = jnp.pad(wg, ((0, D - wg.shape[0]), (0, 0)))
        wsh = jnp.pad(w1[:3] / radius, ((0, 5), (0, 0)))

        new_xyz = jnp.concatenate([nx, ny, nz], axis=2)
        q8 = jnp.pad(new_xyz, ((0, 0), (0, 0), (0, 5))).reshape(B * S, 8)
        feat_flat = _mlp(grouped, q8, wg, b1[None], wsh, w2, b2[None], w3,
                         b3[None], K, min(128, S))

        ind2 = ind[..., 0]
        if i == 0:
            old_ind = ind2
        else:
            prev_s = old_ind.shape[1]
            tab = jnp.broadcast_to(
                old_ind.astype(jnp.float32).reshape(B * prev_s, 1),
                (B * prev_s, 128))
            off = (jnp.arange(B, dtype=jnp.int32) * prev_s)[:, None]
            comp = _sc_gather(jnp.asarray(tab), (ind2 + off).reshape(B * S))
            old_ind = comp[:, 0].reshape(B, S).astype(jnp.int32)

        outs.append(new_xyz)
        outs.append(old_ind.astype(jnp.int64))

        xyz = new_xyz
        feats = feat_flat.reshape(B, S, feat_flat.shape[1])

    final_feats = jnp.transpose(feats, (0, 2, 1))
    return (xyz, final_feats) + tuple(outs)


# speedup vs baseline: 15.0765x; 1.5150x over previous
import functools

import jax
import jax.numpy as jnp
from jax import lax
from jax.experimental import pallas as pl
from jax.experimental.pallas import tpu as pltpu
from jax.experimental.pallas import tpu_sc as plsc

_NPOINT = (2048, 1024, 512)
_RADIUS = (0.2, 0.4, 0.8)
_NSAMPLE = (64, 32, 16)


def _fps_body(x_ref, y_ref, z_ref, xs_ref, ys_ref, zs_ref, ind_ref, nx_ref,
              ny_ref, nz_ref, *, S, N):
    nd8 = N // 8
    X = x_ref[0]
    Y = y_ref[0]
    Z = z_ref[0]
    row = lax.broadcasted_iota(jnp.int32, (8, nd8), 0)
    col = lax.broadcasted_iota(jnp.int32, (8, nd8), 1)
    iota = row * nd8 + col

    def step(i, carry):
        dists, far = carry
        cx = xs_ref[0, 0, far]
        cy = ys_ref[0, 0, far]
        cz = zs_ref[0, 0, far]
        dx = X - cx
        dy = Y - cy
        dz = Z - cz
        d = (dx * dx + dy * dy) + dz * dz
        dists = jnp.minimum(dists, d)
        ind_ref[0, pl.ds(i, 1), :] = far.reshape(1, 1)
        nx_ref[0, pl.ds(i, 1), :] = cx.reshape(1, 1)
        ny_ref[0, pl.ds(i, 1), :] = cy.reshape(1, 1)
        nz_ref[0, pl.ds(i, 1), :] = cz.reshape(1, 1)
        m = jnp.max(dists)
        far2 = jnp.min(jnp.where(dists == m, iota, N))
        return dists, far2

    d0 = jnp.full((8, nd8), 1e10, jnp.float32)
    lax.fori_loop(0, S, step, (d0, jnp.zeros((), jnp.int32)))


def _fps(xp, yp, zp, S):
    B, _, nd8 = xp.shape
    N = 8 * nd8
    body = functools.partial(_fps_body, S=S, N=N)
    in_spec = pl.BlockSpec((1, 8, nd8), lambda b: (b, 0, 0))
    s_spec = pl.BlockSpec((1, 1, N), lambda b: (b, 0, 0),
                          memory_space=pltpu.SMEM)
    out_spec = pl.BlockSpec((1, S, 1), lambda b: (b, 0, 0))
    out_shape = [jax.ShapeDtypeStruct((B, S, 1), jnp.int32)] + [
        jax.ShapeDtypeStruct((B, S, 1), jnp.float32)
    ] * 3
    flat = [a.reshape(B, 1, N) for a in (xp, yp, zp)]
    return pl.pallas_call(
        body,
        grid=(B,),
        in_specs=[in_spec] * 3 + [s_spec] * 3,
        out_specs=[out_spec] * 4,
        out_shape=out_shape,
        compiler_params=pltpu.CompilerParams(
            dimension_semantics=("parallel",)
        ),
    )(xp, yp, zp, *flat)


def _ball_body(qx_ref, qy_ref, qz_ref, xl_ref, yl_ref, zl_ref, idx_ref, *,
               K, N, r2, sblk):
    b = pl.program_id(0)
    qx = qx_ref[0]
    qy = qy_ref[0]
    qz = qz_ref[0]
    X = xl_ref[0]
    Y = yl_ref[0]
    Z = zl_ref[0]
    dx = qx - X
    dy = qy - Y
    dz = qz - Z
    d2 = (dx * dx + dy * dy) + dz * dz
    iota = lax.broadcasted_iota(jnp.int32, (sblk, N), 1)
    order = jnp.where(d2 < r2, iota, N)
    lane_k = lax.broadcasted_iota(jnp.int32, (sblk, K), 1)
    out = jnp.zeros((sblk, K), jnp.int32)
    slot = jnp.zeros((sblk, 1), jnp.int32)
    nch = max(1, N // 1024)
    ch = N // nch
    for c in range(nch):
        oc = order[:, c * ch:(c + 1) * ch]
        ccnt = jnp.sum((oc < N).astype(jnp.int32), axis=1, keepdims=True)
        go0 = jnp.any((ccnt > 0) & (slot < K))

        def body(st, oc=oc, ccnt=ccnt):
            _, prev, tc, slot, out = st
            cur = jnp.min(jnp.where(oc > prev, oc, N), axis=1, keepdims=True)
            valid = (cur < N) & (slot < K)
            out = jnp.where(valid & (lane_k == slot), cur, out)
            slot = slot + valid.astype(jnp.int32)
            tc = tc + (cur < N).astype(jnp.int32)
            go = jnp.any((tc < ccnt) & (slot < K))
            return go, cur, tc, slot, out

        st = (go0, jnp.full((sblk, 1), -1, jnp.int32),
              jnp.zeros((sblk, 1), jnp.int32), slot, out)
        _, _, _, slot, out = lax.while_loop(lambda st: st[0], body, st)
    first = out[:, 0:1]
    idx_ref[0] = jnp.where(lane_k < slot, out, first) + b * N


def _ball(nx, ny, nz, xl, yl, zl, K, radius, sblk):
    B, S, _ = nx.shape
    N = xl.shape[2]
    body = functools.partial(_ball_body, K=K, N=N, r2=radius * radius,
                             sblk=sblk)
    q_spec = pl.BlockSpec((1, sblk, 1), lambda b, s: (b, s, 0))
    l_spec = pl.BlockSpec((1, 1, N), lambda b, s: (b, 0, 0))
    out_spec = pl.BlockSpec((1, sblk, K), lambda b, s: (b, s, 0))
    return pl.pallas_call(
        body,
        grid=(B, S // sblk),
        in_specs=[q_spec] * 3 + [l_spec] * 3,
        out_specs=out_spec,
        out_shape=jax.ShapeDtypeStruct((B, S, K), jnp.int32),
        compiler_params=pltpu.CompilerParams(
            dimension_semantics=("parallel", "parallel")
        ),
    )(nx, ny, nz, xl, yl, zl)


def _sc_gather(table, idx):
    V, D = table.shape
    Bt = idx.shape[0]
    info = plsc.get_sparse_core_info()
    nw = info.num_cores * info.num_subcores
    b_per_w = Bt // nw
    cap = max(8, (65536 // D) // 8 * 8)
    chunk = b_per_w
    while chunk > cap:
        chunk //= 2
    nchunk = b_per_w // chunk
    mesh = plsc.VectorSubcoreMesh(core_axis_name="c", subcore_axis_name="s")

    @functools.partial(
        pl.kernel,
        mesh=mesh,
        out_type=jax.ShapeDtypeStruct((Bt, D), jnp.float32),
        scratch_types=[
            pltpu.VMEM((chunk,), jnp.int32),
            pltpu.VMEM((chunk, D), jnp.float32),
            pltpu.SemaphoreType.DMA,
        ],
    )
    def k(table_hbm, idx_hbm, out_hbm, idx_v, rows_v, sem):
        wid = lax.axis_index("s") * info.num_cores + lax.axis_index("c")
        base = wid * b_per_w
        for i in range(nchunk):
            off = base + i * chunk
            pltpu.sync_copy(idx_hbm.at[pl.ds(off, chunk)], idx_v)
            pltpu.async_copy(table_hbm.at[idx_v], rows_v, sem).wait()
            pltpu.sync_copy(rows_v, out_hbm.at[pl.ds(off, chunk)])

    return k(table, idx)


def _mlp_body(g_ref, q_ref, wg_ref, b1_ref, wsh_ref, w2_ref, b2_ref, w3_ref,
              b3_ref, out_ref, *, K, sblk):
    hp = lax.Precision.HIGHEST
    g = g_ref[...]
    A = jnp.dot(g, wg_ref[...], preferred_element_type=jnp.float32,
                precision=hp)
    shift = jnp.dot(q_ref[...], wsh_ref[...],
                    preferred_element_type=jnp.float32, precision=hp)
    c1 = A.shape[1]
    A = (A.reshape(sblk, K, c1) - shift.reshape(sblk, 1, c1)).reshape(
        sblk * K, c1)
    h = jnp.maximum(A + b1_ref[...], 0.0)
    h = jnp.maximum(
        jnp.dot(h, w2_ref[...], preferred_element_type=jnp.float32,
                precision=hp) + b2_ref[...], 0.0)
    h = jnp.maximum(
        jnp.dot(h, w3_ref[...], preferred_element_type=jnp.float32,
                precision=hp) + b3_ref[...], 0.0)
    cout = h.shape[1]
    out_ref[...] = jnp.max(h.reshape(sblk, K, cout), axis=1)


def _mlp(g, q8, wg, b1, wsh, w2, b2, w3, b3, K, sblk):
    BT, D = g.shape
    BS = BT // K
    c3 = w3.shape[1]
    body = functools.partial(_mlp_body, K=K, sblk=sblk)
    full = lambda a: pl.BlockSpec(a.shape, lambda i: (0,) * a.ndim)
    return pl.pallas_call(
        body,
        grid=(BS // sblk,),
        in_specs=[
            pl.BlockSpec((sblk * K, D), lambda i: (i, 0)),
            pl.BlockSpec((sblk, 8), lambda i: (i, 0)),
            full(wg), full(b1), full(wsh), full(w2), full(b2), full(w3),
            full(b3),
        ],
        out_specs=pl.BlockSpec((sblk, c3), lambda i: (i, 0)),
        out_shape=jax.ShapeDtypeStruct((BS, c3), jnp.float32),
        compiler_params=pltpu.CompilerParams(
            dimension_semantics=("parallel",)
        ),
    )(g, q8, wg, b1, wsh, w2, b2, w3, b3)


def _fold_bn(W, bb, g, be, mu, var):
    s = g * lax.rsqrt(var + 1e-5)
    return W.T * s[None, :], bb * s + be - mu * s


def _pad_cols(a, d):
    return jnp.pad(a, ((0, 0), (0, d - a.shape[1])))


def _round128(n):
    return (n + 127) // 128 * 128


def kernel(input, params):
    B, _, N0 = input.shape
    xyzf = jnp.transpose(input, (0, 2, 1))
    xyz = xyzf[..., :3]
    feats = xyzf[..., 3:]

    outs = []
    old_ind = None
    for i in range(3):
        S, K, radius = _NPOINT[i], _NSAMPLE[i], _RADIUS[i]
        N = xyz.shape[1]
        C = feats.shape[2]

        xp = xyz[..., 0].reshape(B, 8, N // 8)
        yp = xyz[..., 1].reshape(B, 8, N // 8)
        zp = xyz[..., 2].reshape(B, 8, N // 8)
        ind, nx, ny, nz = _fps(xp, yp, zp, S)

        xl = xyz[..., 0].reshape(B, 1, N)
        yl = xyz[..., 1].reshape(B, 1, N)
        zl = xyz[..., 2].reshape(B, 1, N)
        idx = _ball(nx, ny, nz, xl, yl, zl, K, radius, min(128, S))

        D = _round128(3 + C)
        table = _pad_cols(
            jnp.concatenate([xyz, feats], axis=2).reshape(B * N, 3 + C), D)
        grouped = _sc_gather(table, idx.reshape(B * S * K))

        (w1, b1) = _fold_bn(*params[i][0])
        (w2, b2) = _fold_bn(*params[i][1])
        (w3, b3) = _fold_bn(*params[i][2])
        wg = jnp.concatenate([w1[:3] / radius, w1[3:]], axis=0)
        wg = jnp.pad(wg, ((0, D - wg.shape[0]), (0, 0)))
        wsh = jnp.pad(w1[:3] / radius, ((0, 5), (0, 0)))

        new_xyz = jnp.concatenate([nx, ny, nz], axis=2)
        q8 = jnp.pad(new_xyz, ((0, 0), (0, 0), (0, 5))).reshape(B * S, 8)
        feat_flat = _mlp(grouped, q8, wg, b1[None], wsh, w2, b2[None], w3,
                         b3[None], K, min(128, S))

        ind2 = ind[..., 0]
        if i == 0:
            old_ind = ind2
        else:
            prev_s = old_ind.shape[1]
            tab = jnp.broadcast_to(
                old_ind.astype(jnp.float32).reshape(B * prev_s, 1),
                (B * prev_s, 128))
            off = (jnp.arange(B, dtype=jnp.int32) * prev_s)[:, None]
            comp = _sc_gather(jnp.asarray(tab), (ind2 + off).reshape(B * S))
            old_ind = comp[:, 0].reshape(B, S).astype(jnp.int32)

        outs.append(new_xyz)
        outs.append(old_ind.astype(jnp.int64))

        xyz = new_xyz
        feats = feat_flat.reshape(B, S, feat_flat.shape[1])

    final_feats = jnp.transpose(feats, (0, 2, 1))
    return (xyz, final_feats) + tuple(outs)
